# baseline (speedup 1.0000x reference)
import functools

import jax
import jax.numpy as jnp
from jax.experimental import pallas as pl
from jax.experimental.pallas import tpu as pltpu

B, S, D = 4, 8192, 1024
HID = 512
NUM_OUT = 8

SUB = 1024
PER_B = S // SUB
NSUB = B * PER_B
NBUF = 8
INFLIGHT = 4
G = 2
NGRP = NSUB // G
GPB = PER_B // G


def _router_kernel(x_ref, wp_ref, w1_ref, b1_ref, w2_ref, b2_ref,
                   out_ref, bufs, sems):
    def dma(i):
        return pltpu.make_async_copy(
            x_ref.at[i // PER_B, pl.ds((i % PER_B) * SUB, SUB), :],
            bufs.at[i % NBUF],
            sems.at[i % NBUF])

    for i in range(INFLIGHT):
        dma(i).start()

    wp_row = wp_ref[...].reshape(1, D)
    gm, gl, gacc = [], [], []
    for g in range(NGRP):
        xs, ss = [], []
        for u in range(G):
            i = G * g + u
            dma(i).wait()
            if i + INFLIGHT < NSUB:
                dma(i + INFLIGHT).start()
            xb = bufs[i % NBUF]
            xs.append(xb)
            ss.append(jnp.sum(xb * wp_row, axis=1, keepdims=True))
        m_c = jnp.max(ss[0])
        for u in range(1, G):
            m_c = jnp.maximum(m_c, jnp.max(ss[u]))
        ps = [jnp.exp(s - m_c) for s in ss]
        l_c = ps[0].sum()
        acc_c = jnp.dot(ps[0].T, xs[0], preferred_element_type=jnp.float32)
        for u in range(1, G):
            l_c = l_c + ps[u].sum()
            acc_c = acc_c + jnp.dot(ps[u].T, xs[u],
                                    preferred_element_type=jnp.float32)
        gm.append(m_c)
        gl.append(l_c)
        gacc.append(acc_c)

    pooled_rows = []
    for b in range(B):
        ms = gm[b * GPB:(b + 1) * GPB]
        mb = ms[0]
        for mg in ms[1:]:
            mb = jnp.maximum(mb, mg)
        scale = [jnp.exp(mg - mb) for mg in ms]
        lb = sum(sc * lg for sc, lg in zip(scale, gl[b * GPB:(b + 1) * GPB]))
        accb = sum(sc * ag for sc, ag in zip(scale, gacc[b * GPB:(b + 1) * GPB]))
        pooled_rows.append(accb / lb)

    pooled = jnp.concatenate(pooled_rows, axis=0)
    h = jnp.dot(pooled, w1_ref[...], preferred_element_type=jnp.float32)
    h = jnp.maximum(h + b1_ref[...], 0.0)
    logits = jnp.dot(h, w2_ref[...], preferred_element_type=jnp.float32)
    logits = logits + b2_ref[...]

    col = jax.lax.broadcasted_iota(jnp.int32, (B, NUM_OUT), 1)
    m1 = jnp.max(logits, axis=1, keepdims=True)
    i1 = jnp.min(jnp.where(logits == m1, col, NUM_OUT), axis=1, keepdims=True)
    l2 = jnp.where(col == i1, -jnp.inf, logits)
    m2 = jnp.max(l2, axis=1, keepdims=True)
    i2 = jnp.min(jnp.where(l2 == m2, col, NUM_OUT), axis=1, keepdims=True)
    keep = (col == i1) | (col == i2)
    e = jnp.where(keep, jnp.exp(logits - m1), 0.0)
    out_ref[...] = e / jnp.sum(e, axis=1, keepdims=True)


@jax.jit
def kernel(x, w_pool, b_pool, w1, b1, w2, b2):
    del b_pool
    return pl.pallas_call(
        _router_kernel,
        in_specs=[
            pl.BlockSpec(memory_space=pl.ANY),
            pl.BlockSpec((D, 1), lambda: (0, 0)),
            pl.BlockSpec((D, HID), lambda: (0, 0)),
            pl.BlockSpec((1, HID), lambda: (0, 0)),
            pl.BlockSpec((HID, NUM_OUT), lambda: (0, 0)),
            pl.BlockSpec((1, NUM_OUT), lambda: (0, 0)),
        ],
        out_specs=pl.BlockSpec((B, NUM_OUT), lambda: (0, 0)),
        out_shape=jax.ShapeDtypeStruct((B, NUM_OUT), jnp.float32),
        scratch_shapes=[
            pltpu.VMEM((NBUF, SUB, D), jnp.float32),
            pltpu.SemaphoreType.DMA((NBUF,)),
        ],
    )(x, w_pool, w1, b1.reshape(1, HID), w2, b2.reshape(1, NUM_OUT))


# speedup vs baseline: 1.0214x; 1.0214x over previous
import functools

import jax
import jax.numpy as jnp
from jax.experimental import pallas as pl
from jax.experimental.pallas import tpu as pltpu

B, S, D = 4, 8192, 1024
HID = 512
NUM_OUT = 8
CS = 4096
NC = S // CS


def _router_kernel(x_ref, w_pool_ref, w1_ref, b1_ref, w2_ref, b2_ref,
                   out_ref, pooled_ref, m_ref, l_ref):
    b = pl.program_id(0)
    c = pl.program_id(1)

    @pl.when(c == 0)
    def _init():
        m_ref[0] = -jnp.inf
        l_ref[0] = 0.0

    x_blk = x_ref[0]
    wp_row = w_pool_ref[...].reshape(1, D)
    s = jax.lax.dot_general(
        wp_row, x_blk, (((1,), (1,)), ((), ())),
        preferred_element_type=jnp.float32)
    m_c = jnp.max(s)
    m_prev = m_ref[0]
    m_new = jnp.maximum(m_prev, m_c)
    m_ref[0] = m_new
    alpha = jnp.exp(m_prev - m_new)
    p = jnp.exp(s - m_new)
    l_ref[0] = l_ref[0] * alpha + jnp.sum(p)
    acc_c = jnp.dot(p, x_blk, preferred_element_type=jnp.float32)

    @pl.when(c == 0)
    def _first():
        pooled_ref[pl.ds(b, 1), :] = acc_c

    @pl.when(c > 0)
    def _rest():
        pooled_ref[pl.ds(b, 1), :] = pooled_ref[pl.ds(b, 1), :] * alpha + acc_c

    @pl.when(c == NC - 1)
    def _finish_batch():
        pooled_ref[pl.ds(b, 1), :] = pooled_ref[pl.ds(b, 1), :] / l_ref[0]

    @pl.when((b == B - 1) & (c == NC - 1))
    def _mlp():
        pooled = pooled_ref[...]
        h = jnp.dot(pooled, w1_ref[...], preferred_element_type=jnp.float32)
        h = jnp.maximum(h + b1_ref[...], 0.0)
        logits = jnp.dot(h, w2_ref[...], preferred_element_type=jnp.float32)
        logits = logits + b2_ref[...]

        col = jax.lax.broadcasted_iota(jnp.int32, (B, NUM_OUT), 1)
        m1 = jnp.max(logits, axis=1, keepdims=True)
        i1 = jnp.min(jnp.where(logits == m1, col, NUM_OUT), axis=1, keepdims=True)
        l2 = jnp.where(col == i1, -jnp.inf, logits)
        m2 = jnp.max(l2, axis=1, keepdims=True)
        i2 = jnp.min(jnp.where(l2 == m2, col, NUM_OUT), axis=1, keepdims=True)
        keep = (col == i1) | (col == i2)
        e = jnp.where(keep, jnp.exp(logits - m1), 0.0)
        out_ref[...] = e / jnp.sum(e, axis=1, keepdims=True)


@functools.partial(jax.jit, static_argnames=())
def kernel(x, w_pool, b_pool, w1, b1, w2, b2):
    del b_pool
    b1_2d = b1.reshape(1, HID)
    b2_2d = b2.reshape(1, NUM_OUT)
    return pl.pallas_call(
        _router_kernel,
        grid=(B, NC),
        in_specs=[
            pl.BlockSpec((1, CS, D), lambda b, c: (b, c, 0)),
            pl.BlockSpec((D, 1), lambda b, c: (0, 0)),
            pl.BlockSpec((D, HID), lambda b, c: (0, 0)),
            pl.BlockSpec((1, HID), lambda b, c: (0, 0)),
            pl.BlockSpec((HID, NUM_OUT), lambda b, c: (0, 0)),
            pl.BlockSpec((1, NUM_OUT), lambda b, c: (0, 0)),
        ],
        out_specs=pl.BlockSpec((B, NUM_OUT), lambda b, c: (0, 0)),
        out_shape=jax.ShapeDtypeStruct((B, NUM_OUT), jnp.float32),
        scratch_shapes=[
            pltpu.VMEM((B, D), jnp.float32),
            pltpu.SMEM((1,), jnp.float32),
            pltpu.SMEM((1,), jnp.float32),
        ],
    )(x, w_pool, w1, b1_2d, w2, b2_2d)
